# baseline (speedup 1.0000x reference)
import functools

import jax
import jax.numpy as jnp
from jax import lax
from jax.experimental import pallas as pl
from jax.experimental.pallas import tpu as pltpu
from jax.experimental.pallas import tpu_sc as plsc

T, B, D = 2048, 4, 1024
NC, NS, L = 2, 16, 16
NW = NC * NS
TPW = T // NW
CHUNK = 16
NCHUNK = TPW // CHUNK
VECS = D // L


def _sc_body(x_hbm, pos_hbm, out_hbm, x_v, pos_v):
    wid = lax.axis_index("s") * NC + lax.axis_index("c")
    base = wid * TPW

    for c in range(NCHUNK):
        t0 = base + c * CHUNK
        pltpu.sync_copy(x_hbm.at[pl.ds(t0, CHUNK)], x_v)
        pltpu.sync_copy(pos_hbm.at[pl.ds(t0, CHUNK)], pos_v)

        def row_body(t, carry):
            for j in range(VECS):
                p = pos_v[t, pl.ds(j * L, L)]
                for b in range(B):
                    x_v[t, b, pl.ds(j * L, L)] = x_v[t, b, pl.ds(j * L, L)] + p
            return carry

        lax.fori_loop(0, CHUNK, row_body, 0)

        pltpu.sync_copy(x_v, out_hbm.at[pl.ds(t0, CHUNK)])


def kernel(x, pos_table):
    mesh = plsc.VectorSubcoreMesh(core_axis_name="c", subcore_axis_name="s")
    k = functools.partial(
        pl.kernel,
        mesh=mesh,
        out_type=jax.ShapeDtypeStruct((T, B, D), jnp.float32),
        scratch_types=[
            pltpu.VMEM((CHUNK, B, D), jnp.float32),
            pltpu.VMEM((CHUNK, D), jnp.float32),
        ],
    )(_sc_body)
    return k(x, pos_table)
